# baseline (speedup 1.0000x reference)
import functools

import jax
import jax.numpy as jnp
from jax import lax
from jax.experimental import pallas as pl
from jax.experimental.pallas import tpu as pltpu
from jax.experimental.pallas import tpu_sc as plsc

_SEQ_BLOCK = 2048


def _tc_body(x_ref, emb_ref, out_ref):
    out_ref[0] = x_ref[0] + emb_ref[...]


def _tc_add(x, embedding):
    batch, seq_len, dim = x.shape
    grid = (seq_len // _SEQ_BLOCK, batch)
    return pl.pallas_call(
        _tc_body,
        grid=grid,
        in_specs=[
            pl.BlockSpec((1, _SEQ_BLOCK, dim), lambda s, b: (b, s, 0)),
            pl.BlockSpec((_SEQ_BLOCK, dim), lambda s, b: (s, 0)),
        ],
        out_specs=pl.BlockSpec((1, _SEQ_BLOCK, dim), lambda s, b: (b, s, 0)),
        out_shape=jax.ShapeDtypeStruct(x.shape, x.dtype),
        compiler_params=pltpu.CompilerParams(
            vmem_limit_bytes=64 * 1024 * 1024,
        ),
    )(x, embedding)


_CHUNK = 16384


def _sc_add(x_flat, emb_flat, n_batch, seq_elems):
    info = plsc.get_sparse_core_info()
    nw = info.num_cores * info.num_subcores
    per_w = seq_elems // nw
    n_chunks = per_w // _CHUNK
    mesh = plsc.VectorSubcoreMesh(core_axis_name="c", subcore_axis_name="s")

    @functools.partial(
        pl.kernel,
        mesh=mesh,
        out_type=jax.ShapeDtypeStruct(x_flat.shape, x_flat.dtype),
        scratch_types=[
            pltpu.VMEM((_CHUNK,), jnp.float32),
            pltpu.VMEM((_CHUNK,), jnp.float32),
        ],
    )
    def k(x_hbm, emb_hbm, out_hbm, xbuf, ebuf):
        wid = lax.axis_index("s") * info.num_cores + lax.axis_index("c")
        base_w = wid * per_w

        def chunk_body(ci, carry):
            off = pl.multiple_of(base_w + ci * _CHUNK, 8)
            pltpu.sync_copy(emb_hbm.at[pl.ds(off, _CHUNK)], ebuf)

            def batch_body(b, carry):
                xoff = pl.multiple_of(b * seq_elems + base_w + ci * _CHUNK, 8)
                pltpu.sync_copy(x_hbm.at[pl.ds(xoff, _CHUNK)], xbuf)

                def add_body(i, carry):
                    sl = pl.ds(i * 16, 16)
                    xbuf[sl] = xbuf[sl] + ebuf[sl]
                    return carry

                lax.fori_loop(0, _CHUNK // 16, add_body, 0)
                pltpu.sync_copy(xbuf, out_hbm.at[pl.ds(xoff, _CHUNK)])
                return carry

            lax.fori_loop(0, n_batch, batch_body, 0)
            return carry

        lax.fori_loop(0, n_chunks, chunk_body, 0)

    return k(x_flat, emb_flat)


def kernel(x, embedding):
    batch, seq_len, dim = x.shape
    out_flat = _sc_add(
        x.reshape(-1), embedding.reshape(-1), batch, seq_len * dim
    )
    return out_flat.reshape(x.shape)


# speedup vs baseline: 1.2574x; 1.2574x over previous
import functools

import jax
import jax.numpy as jnp
from jax import lax
from jax.experimental import pallas as pl
from jax.experimental.pallas import tpu as pltpu
from jax.experimental.pallas import tpu_sc as plsc

_SEQ_BLOCK = 2048


def _tc_body(x_ref, emb_ref, out_ref):
    out_ref[0] = x_ref[0] + emb_ref[...]


def _tc_add(x, embedding):
    batch, seq_len, dim = x.shape
    grid = (seq_len // _SEQ_BLOCK, batch)
    return pl.pallas_call(
        _tc_body,
        grid=grid,
        in_specs=[
            pl.BlockSpec((1, _SEQ_BLOCK, dim), lambda s, b: (b, s, 0)),
            pl.BlockSpec((_SEQ_BLOCK, dim), lambda s, b: (s, 0)),
        ],
        out_specs=pl.BlockSpec((1, _SEQ_BLOCK, dim), lambda s, b: (b, s, 0)),
        out_shape=jax.ShapeDtypeStruct(x.shape, x.dtype),
        compiler_params=pltpu.CompilerParams(
            vmem_limit_bytes=64 * 1024 * 1024,
        ),
    )(x, embedding)


_CHUNK = 16384


_N_XBUF = 3
_N_EBUF = 2


def _sc_add(x_flat, emb_flat, n_batch, seq_elems):
    info = plsc.get_sparse_core_info()
    nw = info.num_cores * info.num_subcores
    per_w = seq_elems // nw
    n_chunks = per_w // _CHUNK
    n_steps = n_chunks * n_batch
    mesh = plsc.VectorSubcoreMesh(core_axis_name="c", subcore_axis_name="s")

    @functools.partial(
        pl.kernel,
        mesh=mesh,
        out_type=jax.ShapeDtypeStruct(x_flat.shape, x_flat.dtype),
        scratch_types=[
            [pltpu.VMEM((_CHUNK,), jnp.float32) for _ in range(_N_XBUF)],
            [pltpu.VMEM((_CHUNK,), jnp.float32) for _ in range(_N_EBUF)],
            [pltpu.SemaphoreType.DMA for _ in range(_N_XBUF)],
            [pltpu.SemaphoreType.DMA for _ in range(_N_EBUF)],
            [pltpu.SemaphoreType.DMA for _ in range(_N_XBUF)],
        ],
    )
    def k(x_hbm, emb_hbm, out_hbm, xbufs, ebufs, xsems, esems, osems):
        wid = lax.axis_index("s") * info.num_cores + lax.axis_index("c")
        base_w = wid * per_w

        def xoff(step):
            ci, b = divmod(step, n_batch)
            return pl.multiple_of(base_w + b * seq_elems + ci * _CHUNK, 8)

        def eoff(ci):
            return pl.multiple_of(base_w + ci * _CHUNK, 8)

        def start_xload(step):
            j = step % _N_XBUF
            return pltpu.async_copy(
                x_hbm.at[pl.ds(xoff(step), _CHUNK)], xbufs[j], xsems[j]
            )

        def start_eload(ci):
            j = ci % _N_EBUF
            return pltpu.async_copy(
                emb_hbm.at[pl.ds(eoff(ci), _CHUNK)], ebufs[j], esems[j]
            )

        def start_store(step):
            j = step % _N_XBUF
            return pltpu.async_copy(
                xbufs[j], out_hbm.at[pl.ds(xoff(step), _CHUNK)], osems[j]
            )

        loads = {0: start_xload(0)}
        eloads = {0: start_eload(0)}
        stores = {}
        for i in range(n_steps):
            ci, b = divmod(i, n_batch)
            nxt = i + 1
            if nxt < n_steps:
                prev = nxt - _N_XBUF
                if prev in stores:
                    stores.pop(prev).wait()
                loads[nxt] = start_xload(nxt)
                nci = nxt // n_batch
                if nci != ci and nci not in eloads:
                    eloads[nci] = start_eload(nci)
            if b == 0:
                eloads.pop(ci).wait()
            loads.pop(i).wait()
            xb = xbufs[i % _N_XBUF]
            eb = ebufs[ci % _N_EBUF]

            def add_body(j, carry):
                sl = pl.ds(j * 16, 16)
                xb[sl] = xb[sl] + eb[sl]
                return carry

            lax.fori_loop(0, _CHUNK // 16, add_body, 0)
            stores[i] = start_store(i)
        for s in stores.values():
            s.wait()

    return k(x_flat, emb_flat)


def kernel(x, embedding):
    batch, seq_len, dim = x.shape
    out_flat = _sc_add(
        x.reshape(-1), embedding.reshape(-1), batch, seq_len * dim
    )
    return out_flat.reshape(x.shape)


# speedup vs baseline: 1.8464x; 1.4685x over previous
import functools

import jax
import jax.numpy as jnp
from jax import lax
from jax.experimental import pallas as pl
from jax.experimental.pallas import tpu as pltpu
from jax.experimental.pallas import tpu_sc as plsc

_SEQ_BLOCK = 2048


def _tc_body(x_ref, emb_ref, out_ref):
    out_ref[0] = x_ref[0] + emb_ref[...]


def _tc_add(x, embedding):
    batch, seq_len, dim = x.shape
    grid = (seq_len // _SEQ_BLOCK, batch)
    return pl.pallas_call(
        _tc_body,
        grid=grid,
        in_specs=[
            pl.BlockSpec((1, _SEQ_BLOCK, dim), lambda s, b: (b, s, 0)),
            pl.BlockSpec((_SEQ_BLOCK, dim), lambda s, b: (s, 0)),
        ],
        out_specs=pl.BlockSpec((1, _SEQ_BLOCK, dim), lambda s, b: (b, s, 0)),
        out_shape=jax.ShapeDtypeStruct(x.shape, x.dtype),
        compiler_params=pltpu.CompilerParams(
            vmem_limit_bytes=64 * 1024 * 1024,
        ),
    )(x, embedding)


_CHUNK = 16384


_N_XBUF = 3
_N_EBUF = 2


def _sc_add(x_flat, emb_flat, n_batch, seq_elems):
    info = plsc.get_sparse_core_info()
    nw = info.num_cores * info.num_subcores
    per_w = seq_elems // nw
    n_chunks = per_w // _CHUNK
    n_steps = n_chunks * n_batch
    mesh = plsc.VectorSubcoreMesh(core_axis_name="c", subcore_axis_name="s")

    @functools.partial(
        pl.kernel,
        mesh=mesh,
        out_type=jax.ShapeDtypeStruct(x_flat.shape, x_flat.dtype),
        scratch_types=[
            [pltpu.VMEM((_CHUNK,), jnp.float32) for _ in range(_N_XBUF)],
            [pltpu.VMEM((_CHUNK,), jnp.float32) for _ in range(_N_EBUF)],
            [pltpu.SemaphoreType.DMA for _ in range(_N_XBUF)],
            [pltpu.SemaphoreType.DMA for _ in range(_N_EBUF)],
            [pltpu.SemaphoreType.DMA for _ in range(_N_XBUF)],
        ],
    )
    def k(x_hbm, emb_hbm, out_hbm, xbufs, ebufs, xsems, esems, osems):
        wid = lax.axis_index("s") * info.num_cores + lax.axis_index("c")
        base_w = wid * per_w

        def xoff(step):
            ci, b = divmod(step, n_batch)
            return pl.multiple_of(base_w + b * seq_elems + ci * _CHUNK, 8)

        def eoff(ci):
            return pl.multiple_of(base_w + ci * _CHUNK, 8)

        def start_xload(step):
            j = step % _N_XBUF
            return pltpu.async_copy(
                x_hbm.at[pl.ds(xoff(step), _CHUNK)], xbufs[j], xsems[j]
            )

        def start_eload(ci):
            j = ci % _N_EBUF
            return pltpu.async_copy(
                emb_hbm.at[pl.ds(eoff(ci), _CHUNK)], ebufs[j], esems[j]
            )

        def start_store(step):
            j = step % _N_XBUF
            return pltpu.async_copy(
                xbufs[j], out_hbm.at[pl.ds(xoff(step), _CHUNK)], osems[j]
            )

        loads = {0: start_xload(0)}
        eloads = {0: start_eload(0)}
        stores = {}
        for i in range(n_steps):
            ci, b = divmod(i, n_batch)
            nxt = i + 1
            if nxt < n_steps:
                prev = nxt - _N_XBUF
                if prev in stores:
                    stores.pop(prev).wait()
                loads[nxt] = start_xload(nxt)
                nci = nxt // n_batch
                if nci != ci and nci not in eloads:
                    eloads[nci] = start_eload(nci)
            if b == 0:
                eloads.pop(ci).wait()
            loads.pop(i).wait()
            xb = xbufs[i % _N_XBUF]
            eb = ebufs[ci % _N_EBUF]

            @plsc.parallel_loop(0, _CHUNK, step=16, unroll=8)
            def _(j):
                sl = pl.ds(pl.multiple_of(j, 16), 16)
                xb[sl] = xb[sl] + eb[sl]
            stores[i] = start_store(i)
        for s in stores.values():
            s.wait()

    return k(x_flat, emb_flat)


def kernel(x, embedding):
    batch, seq_len, dim = x.shape
    out_flat = _sc_add(
        x.reshape(-1), embedding.reshape(-1), batch, seq_len * dim
    )
    return out_flat.reshape(x.shape)


# speedup vs baseline: 5.2158x; 2.8248x over previous
import functools

import jax
import jax.numpy as jnp
from jax import lax
from jax.experimental import pallas as pl
from jax.experimental.pallas import tpu as pltpu
from jax.experimental.pallas import tpu_sc as plsc

_SEQ_BLOCK = 2048


def _tc_body(x_ref, emb_ref, out_ref):
    out_ref[0] = x_ref[0] + emb_ref[...]


def _tc_add(x, embedding):
    batch, seq_len, dim = x.shape
    grid = (seq_len // _SEQ_BLOCK, batch)
    return pl.pallas_call(
        _tc_body,
        grid=grid,
        in_specs=[
            pl.BlockSpec((1, _SEQ_BLOCK, dim), lambda s, b: (b, s, 0)),
            pl.BlockSpec((_SEQ_BLOCK, dim), lambda s, b: (s, 0)),
        ],
        out_specs=pl.BlockSpec((1, _SEQ_BLOCK, dim), lambda s, b: (b, s, 0)),
        out_shape=jax.ShapeDtypeStruct(x.shape, x.dtype),
        compiler_params=pltpu.CompilerParams(
            vmem_limit_bytes=64 * 1024 * 1024,
        ),
    )(x, embedding)


_CHUNK_ROWS = 16
_N_XBUF = 3
_N_EBUF = 2


def _sc_add(x, embedding):
    n_batch, seq_len, dim = x.shape
    info = plsc.get_sparse_core_info()
    nw = info.num_cores * info.num_subcores
    rows_w = seq_len // nw
    n_chunks = rows_w // _CHUNK_ROWS
    n_steps = n_chunks * n_batch
    mesh = plsc.VectorSubcoreMesh(core_axis_name="c", subcore_axis_name="s")

    @functools.partial(
        pl.kernel,
        mesh=mesh,
        out_type=jax.ShapeDtypeStruct(x.shape, x.dtype),
        scratch_types=[
            [pltpu.VMEM((_CHUNK_ROWS, dim), jnp.float32) for _ in range(_N_XBUF)],
            [pltpu.VMEM((_CHUNK_ROWS, dim), jnp.float32) for _ in range(_N_EBUF)],
            [pltpu.SemaphoreType.DMA for _ in range(_N_XBUF)],
            [pltpu.SemaphoreType.DMA for _ in range(_N_EBUF)],
            [pltpu.SemaphoreType.DMA for _ in range(_N_XBUF)],
        ],
    )
    def k(x_hbm, emb_hbm, out_hbm, xbufs, ebufs, xsems, esems, osems):
        wid = lax.axis_index("s") * info.num_cores + lax.axis_index("c")
        base_row = wid * rows_w

        def row0(ci):
            return pl.multiple_of(base_row + ci * _CHUNK_ROWS, 8)

        def start_xload(step):
            ci, b = divmod(step, n_batch)
            j = step % _N_XBUF
            return pltpu.async_copy(
                x_hbm.at[b, pl.ds(row0(ci), _CHUNK_ROWS), :], xbufs[j], xsems[j]
            )

        def start_eload(ci):
            j = ci % _N_EBUF
            return pltpu.async_copy(
                emb_hbm.at[pl.ds(row0(ci), _CHUNK_ROWS), :], ebufs[j], esems[j]
            )

        def start_store(step):
            ci, b = divmod(step, n_batch)
            j = step % _N_XBUF
            return pltpu.async_copy(
                xbufs[j], out_hbm.at[b, pl.ds(row0(ci), _CHUNK_ROWS), :], osems[j]
            )

        loads = {0: start_xload(0)}
        eloads = {0: start_eload(0)}
        stores = {}
        for i in range(n_steps):
            ci, b = divmod(i, n_batch)
            nxt = i + 1
            if nxt < n_steps:
                prev = nxt - _N_XBUF
                if prev in stores:
                    stores.pop(prev).wait()
                loads[nxt] = start_xload(nxt)
                nci = nxt // n_batch
                if nci != ci and nci not in eloads:
                    eloads[nci] = start_eload(nci)
            if b == 0:
                eloads.pop(ci).wait()
            loads.pop(i).wait()
            xb = xbufs[i % _N_XBUF]
            eb = ebufs[ci % _N_EBUF]

            def row_body(r, carry):
                @plsc.parallel_loop(0, dim, step=16, unroll=8)
                def _(j):
                    sl = pl.ds(pl.multiple_of(j, 16), 16)
                    xb[r, sl] = xb[r, sl] + eb[r, sl]

                return carry

            lax.fori_loop(0, _CHUNK_ROWS, row_body, 0)

            stores[i] = start_store(i)
        for s in stores.values():
            s.wait()

    return k(x, embedding)


def kernel(x, embedding):
    return _sc_add(x, embedding)
